# baseline (speedup 1.0000x reference)
import jax
import jax.numpy as jnp
from jax.experimental import pallas as pl
from jax.experimental.pallas import tpu as pltpu


_BLOCK_ROWS = 1024


def _identity_copy_kernel(src_ref, dst_ref):
    dst_ref[...] = src_ref[...]


def kernel(z, embedding):
    del embedding
    rows = z.shape[0] * z.shape[1]
    z2 = z.reshape(rows, z.shape[2])
    out = pl.pallas_call(
        _identity_copy_kernel,
        grid=(rows // _BLOCK_ROWS,),
        in_specs=[pl.BlockSpec((_BLOCK_ROWS, z2.shape[1]), lambda i: (i, 0))],
        out_specs=pl.BlockSpec((_BLOCK_ROWS, z2.shape[1]), lambda i: (i, 0)),
        out_shape=jax.ShapeDtypeStruct(z2.shape, z2.dtype),
        compiler_params=pltpu.CompilerParams(
            dimension_semantics=("parallel",),
        ),
    )(z2)
    return out.reshape(z.shape)


# speedup vs baseline: 1.2673x; 1.2673x over previous
import jax
import jax.numpy as jnp
from jax.experimental import pallas as pl
from jax.experimental.pallas import tpu as pltpu


_BLOCK_ROWS = 2048


def _identity_copy_kernel(src_ref, dst_ref):
    dst_ref[...] = src_ref[...]


def kernel(z, embedding):
    del embedding
    rows = z.shape[0] * z.shape[1]
    z2 = z.reshape(rows, z.shape[2])
    out = pl.pallas_call(
        _identity_copy_kernel,
        grid=(rows // _BLOCK_ROWS,),
        in_specs=[pl.BlockSpec((_BLOCK_ROWS, z2.shape[1]), lambda i: (i, 0))],
        out_specs=pl.BlockSpec((_BLOCK_ROWS, z2.shape[1]), lambda i: (i, 0)),
        out_shape=jax.ShapeDtypeStruct(z2.shape, z2.dtype),
        compiler_params=pltpu.CompilerParams(
            dimension_semantics=("parallel",),
        ),
    )(z2)
    return out.reshape(z.shape)


# speedup vs baseline: 1.5463x; 1.2201x over previous
import jax
import jax.numpy as jnp
from jax.experimental import pallas as pl
from jax.experimental.pallas import tpu as pltpu


_BLOCK_ROWS = 4096


def _identity_copy_kernel(src_ref, dst_ref):
    dst_ref[...] = src_ref[...]


def kernel(z, embedding):
    del embedding
    rows = z.shape[0] * z.shape[1]
    z2 = z.reshape(rows, z.shape[2])
    out = pl.pallas_call(
        _identity_copy_kernel,
        grid=(rows // _BLOCK_ROWS,),
        in_specs=[pl.BlockSpec((_BLOCK_ROWS, z2.shape[1]), lambda i: (i, 0))],
        out_specs=pl.BlockSpec((_BLOCK_ROWS, z2.shape[1]), lambda i: (i, 0)),
        out_shape=jax.ShapeDtypeStruct(z2.shape, z2.dtype),
        compiler_params=pltpu.CompilerParams(
            dimension_semantics=("parallel",),
        ),
    )(z2)
    return out.reshape(z.shape)


# speedup vs baseline: 1.5647x; 1.0119x over previous
import jax
import jax.numpy as jnp
from jax.experimental import pallas as pl
from jax.experimental.pallas import tpu as pltpu


_N_CHUNKS = 4


def _identity_copy_kernel(src_ref, dst_ref, buf_ref, in_sems, out_sems):
    for i in range(_N_CHUNKS):
        pltpu.make_async_copy(src_ref.at[i], buf_ref.at[i], in_sems.at[i]).start()
    for i in range(_N_CHUNKS):
        pltpu.make_async_copy(src_ref.at[i], buf_ref.at[i], in_sems.at[i]).wait()
        pltpu.make_async_copy(buf_ref.at[i], dst_ref.at[i], out_sems.at[i]).start()
    for i in range(_N_CHUNKS):
        pltpu.make_async_copy(buf_ref.at[i], dst_ref.at[i], out_sems.at[i]).wait()


def kernel(z, embedding):
    del embedding
    rows = z.shape[0] * z.shape[1]
    zc = z.reshape(_N_CHUNKS, rows // _N_CHUNKS, z.shape[2])
    out = pl.pallas_call(
        _identity_copy_kernel,
        out_shape=jax.ShapeDtypeStruct(zc.shape, zc.dtype),
        in_specs=[pl.BlockSpec(memory_space=pl.ANY)],
        out_specs=pl.BlockSpec(memory_space=pl.ANY),
        scratch_shapes=[
            pltpu.VMEM(zc.shape, zc.dtype),
            pltpu.SemaphoreType.DMA((_N_CHUNKS,)),
            pltpu.SemaphoreType.DMA((_N_CHUNKS,)),
        ],
    )(zc)
    return out.reshape(z.shape)
